# Initial kernel scaffold; baseline (speedup 1.0000x reference)
import functools

import numpy as np
import jax
import jax.numpy as jnp
from jax import lax
from jax.experimental import pallas as pl
from jax.experimental.pallas import tpu as pltpu
from jax.experimental.pallas import tpu_sc as plsc

_OH, _OW, _G = 7, 7, 2
_SCALE = 0.25
_B, _C, _H, _W = 2, 128, 200, 200
_N = 1024

_PTS = _OH * _OW * _G * _G
_PPTS = 224
_NCH = _PPTS // 16
_ROWS = 4 * _PPTS
_TBL = _B * _H * _W + 208
_NC, _NS = 2, 16
_NWORK = _NC * _NS
_RPW = _N // _NWORK
_OUTF = _C * _OH * _OW


def _point_consts():
    ay = np.zeros(_PPTS, np.float32)
    ax = np.zeros(_PPTS, np.float32)
    for p in range(_PTS):
        b, s = divmod(p, _G * _G)
        oh, ow = divmod(b, _OW)
        gy, gx = divmod(s, _G)
        ay[p] = (oh + (gy + 0.5) / _G) / _OH - 0.5
        ax[p] = (ow + (gx + 0.5) / _G) / _OW - 0.5
    ay[_PTS:] = ay[_PTS - 1]
    ax[_PTS:] = ax[_PTS - 1]
    return ay, ax

_AY_CONST, _AX_CONST = _point_consts()


def _sc_body(tbl_hbm, par_hbm, ay_hbm, ax_hbm, out_hbm,
             par_v, ay_v, ax_v, idx_v, w_v, g_v, out_v, sem):
    wid = lax.axis_index("s") * _NC + lax.axis_index("c")
    pltpu.sync_copy(ay_hbm, ay_v)
    pltpu.sync_copy(ax_hbm, ax_v)
    pltpu.sync_copy(par_hbm.at[pl.ds(wid * (_RPW * 8), _RPW * 8)], par_v)

    lanes = lax.iota(jnp.int32, 16)

    def splat_par(j, k):
        return plsc.load_gather(par_v, [jnp.full((16,), j * 8 + k, jnp.int32)])

    def roi_body(j, _):
        cx = splat_par(j, 0)
        cy = splat_par(j, 1)
        rw = splat_par(j, 2)
        rh = splat_par(j, 3)
        cs = splat_par(j, 4)
        sn = splat_par(j, 5)
        base = splat_par(j, 6).astype(jnp.int32)

        for c in range(_NCH):
            ay = ay_v[pl.ds(c * 16, 16)]
            ax = ax_v[pl.ds(c * 16, 16)]
            yy = rh * ay
            xx = rw * ax
            x = xx * cs - yy * sn + cx
            y = xx * sn + yy * cs + cy
            valid = ((y > -1.0) & (y < float(_H))
                     & (x > -1.0) & (x < float(_W)))
            xc = jnp.minimum(jnp.maximum(x, 0.0), float(_W - 1))
            yc = jnp.minimum(jnp.maximum(y, 0.0), float(_H - 1))
            x0 = xc.astype(jnp.int32)
            y0 = yc.astype(jnp.int32)
            lx = xc - x0.astype(jnp.float32)
            ly = yc - y0.astype(jnp.float32)
            hx = 1.0 - lx
            hy = 1.0 - ly
            vm = jnp.where(valid, 0.25, 0.0)
            r00 = base + y0 * _W + x0
            half = c // 7
            col = (c % 7) * 16
            ws = (hy * hx * vm, hy * lx * vm, ly * hx * vm, ly * lx * vm)
            rs = (r00, r00 + 1, r00 + _W, r00 + _W + 1)
            for k in range(4):
                idx_v[2 * k + half, pl.ds(col, 16)] = rs[k]
                w_v[pl.ds(k * _PPTS + c * 16, 16)] = ws[k]

        copies = [
            pltpu.async_copy(tbl_hbm.at[idx_v.at[kk]],
                             g_v.at[pl.ds(kk * 112, 112)], sem)
            for kk in range(8)
        ]
        for cp in copies:
            cp.wait()

        def bin_body(b, _):
            accs = None
            for s in range(4):
                for k in range(4):
                    r = k * _PPTS + 4 * b + s
                    wspl = plsc.load_gather(w_v, [jnp.full((16,), r, jnp.int32)])
                    terms = [wspl * g_v[r, pl.ds(ch * 16, 16)]
                             for ch in range(8)]
                    if accs is None:
                        accs = terms
                    else:
                        accs = [a + t for a, t in zip(accs, terms)]
            for ch in range(8):
                sidx = (lanes + ch * 16) * (_OH * _OW) + b
                plsc.store_scatter(out_v, [sidx], accs[ch])
            return _

        lax.fori_loop(0, _OH * _OW, bin_body, None)
        pltpu.sync_copy(out_v, out_hbm.at[wid * _RPW + j])
        return _

    lax.fori_loop(0, _RPW, roi_body, None)


@jax.jit
def _roi_align_sc(tbl, params, ayc, axc):
    mesh = plsc.VectorSubcoreMesh(core_axis_name="c", subcore_axis_name="s")
    f = functools.partial(
        pl.kernel,
        out_type=jax.ShapeDtypeStruct((_N, _OUTF), jnp.float32),
        mesh=mesh,
        scratch_types=[
            pltpu.VMEM((_RPW * 8,), jnp.float32),
            pltpu.VMEM((_PPTS,), jnp.float32),
            pltpu.VMEM((_PPTS,), jnp.float32),
            pltpu.VMEM((8, 112), jnp.int32),
            pltpu.VMEM((_ROWS,), jnp.float32),
            pltpu.VMEM((_ROWS, _C), jnp.float32),
            pltpu.VMEM((_OUTF,), jnp.float32),
            pltpu.SemaphoreType.DMA,
        ],
    )(_sc_body)
    return f(tbl, params, ayc, axc)


def kernel(inputs, rois):
    tbl = jnp.transpose(inputs, (0, 2, 3, 1)).reshape(_B * _H * _W, _C)
    tbl = jnp.concatenate([tbl, jnp.zeros((_TBL - _B * _H * _W, _C),
                                          jnp.float32)], axis=0)
    cx = rois[:, 1] * _SCALE
    cy = rois[:, 2] * _SCALE
    rw = jnp.maximum(rois[:, 3] * _SCALE, 1.0)
    rh = jnp.maximum(rois[:, 4] * _SCALE, 1.0)
    cs = jnp.cos(rois[:, 5])
    sn = jnp.sin(rois[:, 5])
    base = rois[:, 0] * float(_H * _W)
    zero = jnp.zeros_like(cx)
    params = jnp.stack([cx, cy, rw, rh, cs, sn, base, zero], 1).reshape(-1)
    out = _roi_align_sc(tbl, params,
                        jnp.asarray(_AY_CONST), jnp.asarray(_AX_CONST))
    return out.reshape(_N, _C, _OH, _OW)


# speedup vs baseline: 29.8583x; 29.8583x over previous
import functools

import numpy as np
import jax
import jax.numpy as jnp
from jax import lax
from jax.experimental import pallas as pl
from jax.experimental.pallas import tpu as pltpu
from jax.experimental.pallas import tpu_sc as plsc

_OH, _OW, _G = 7, 7, 2
_SCALE = 0.25
_B, _C, _H, _W = 2, 128, 200, 200
_N = 1024

_PTS = _OH * _OW * _G * _G
_PPTS = 224
_NCH = _PPTS // 16
_ROWS = 4 * _PPTS
_TBL = _B * _H * _W + 208
_NC, _NS = 2, 16
_NWORK = _NC * _NS
_RPW = _N // _NWORK
_OUTF = _C * _OH * _OW


def _point_consts():
    ay = np.zeros(_PPTS, np.float32)
    ax = np.zeros(_PPTS, np.float32)
    for p in range(_PTS):
        b, s = divmod(p, _G * _G)
        oh, ow = divmod(b, _OW)
        gy, gx = divmod(s, _G)
        ay[p] = (oh + (gy + 0.5) / _G) / _OH - 0.5
        ax[p] = (ow + (gx + 0.5) / _G) / _OW - 0.5
    ay[_PTS:] = ay[_PTS - 1]
    ax[_PTS:] = ax[_PTS - 1]
    return ay, ax

_AY_CONST, _AX_CONST = _point_consts()


def _sc_body(tbl_hbm, par_hbm, ay_hbm, ax_hbm, out_hbm,
             par_v, ay_v, ax_v, idx_v, w_v, g_v, out_v, sem):
    wid = lax.axis_index("s") * _NC + lax.axis_index("c")
    pltpu.sync_copy(ay_hbm, ay_v)
    pltpu.sync_copy(ax_hbm, ax_v)
    pltpu.sync_copy(par_hbm.at[pl.ds(wid * (_RPW * 8), _RPW * 8)], par_v)

    lanes = lax.iota(jnp.int32, 16)

    def splat_par(j, k):
        return plsc.load_gather(par_v, [jnp.full((16,), j * 8 + k, jnp.int32)])

    def roi_body(j, _):
        cx = splat_par(j, 0)
        cy = splat_par(j, 1)
        rw = splat_par(j, 2)
        rh = splat_par(j, 3)
        cs = splat_par(j, 4)
        sn = splat_par(j, 5)
        base = splat_par(j, 6).astype(jnp.int32)

        for c in range(_NCH):
            ay = ay_v[pl.ds(c * 16, 16)]
            ax = ax_v[pl.ds(c * 16, 16)]
            yy = rh * ay
            xx = rw * ax
            x = xx * cs - yy * sn + cx
            y = xx * sn + yy * cs + cy
            valid = ((y > -1.0) & (y < float(_H))
                     & (x > -1.0) & (x < float(_W)))
            xc = jnp.minimum(jnp.maximum(x, 0.0), float(_W - 1))
            yc = jnp.minimum(jnp.maximum(y, 0.0), float(_H - 1))
            x0 = xc.astype(jnp.int32)
            y0 = yc.astype(jnp.int32)
            lx = xc - x0.astype(jnp.float32)
            ly = yc - y0.astype(jnp.float32)
            hx = 1.0 - lx
            hy = 1.0 - ly
            vm = jnp.where(valid, 0.25, 0.0)
            r00 = base + y0 * _W + x0
            half = c // 7
            col = (c % 7) * 16
            ws = (hy * hx * vm, hy * lx * vm, ly * hx * vm, ly * lx * vm)
            rs = (r00, r00 + 1, r00 + _W, r00 + _W + 1)
            for k in range(4):
                idx_v[2 * k + half, pl.ds(col, 16)] = rs[k]
                w_v[pl.ds(k * _PPTS + c * 16, 16)] = ws[k]

        copies = [
            pltpu.async_copy(tbl_hbm.at[idx_v.at[kk]],
                             g_v.at[pl.ds(kk * 112, 112)], sem)
            for kk in range(8)
        ]
        for cp in copies:
            cp.wait()

        def bin_body(b, _):
            accs = None
            for s in range(4):
                for k in range(4):
                    r = k * _PPTS + 4 * b + s
                    wspl = plsc.load_gather(w_v, [jnp.full((16,), r, jnp.int32)])
                    terms = [wspl * g_v[r, pl.ds(ch * 16, 16)]
                             for ch in range(8)]
                    if accs is None:
                        accs = terms
                    else:
                        accs = [a + t for a, t in zip(accs, terms)]
            for ch in range(8):
                sidx = (lanes + ch * 16) * (_OH * _OW) + b
                plsc.store_scatter(out_v, [sidx], accs[ch])
            return _

        lax.fori_loop(0, _OH * _OW, bin_body, None)
        pltpu.sync_copy(out_v, out_hbm.at[wid * _RPW + j])
        return _

    lax.fori_loop(0, _RPW, roi_body, None)


@jax.jit
def _roi_align_sc(tbl, params, ayc, axc):
    mesh = plsc.VectorSubcoreMesh(core_axis_name="c", subcore_axis_name="s")
    f = functools.partial(
        pl.kernel,
        out_type=jax.ShapeDtypeStruct((_N, _OUTF), jnp.float32),
        mesh=mesh,
        compiler_params=pltpu.CompilerParams(needs_layout_passes=False),
        scratch_types=[
            pltpu.VMEM((_RPW * 8,), jnp.float32),
            pltpu.VMEM((_PPTS,), jnp.float32),
            pltpu.VMEM((_PPTS,), jnp.float32),
            pltpu.VMEM((8, 112), jnp.int32),
            pltpu.VMEM((_ROWS,), jnp.float32),
            pltpu.VMEM((_ROWS, _C), jnp.float32),
            pltpu.VMEM((_OUTF,), jnp.float32),
            pltpu.SemaphoreType.DMA,
        ],
    )(_sc_body)
    return f(tbl, params, ayc, axc)


def kernel(inputs, rois):
    tbl = jnp.transpose(inputs, (0, 2, 3, 1)).reshape(_B * _H * _W, _C)
    tbl = jnp.concatenate([tbl, jnp.zeros((_TBL - _B * _H * _W, _C),
                                          jnp.float32)], axis=0)
    cx = rois[:, 1] * _SCALE
    cy = rois[:, 2] * _SCALE
    rw = jnp.maximum(rois[:, 3] * _SCALE, 1.0)
    rh = jnp.maximum(rois[:, 4] * _SCALE, 1.0)
    cs = jnp.cos(rois[:, 5])
    sn = jnp.sin(rois[:, 5])
    base = rois[:, 0] * float(_H * _W)
    zero = jnp.zeros_like(cx)
    params = jnp.stack([cx, cy, rw, rh, cs, sn, base, zero], 1).reshape(-1)
    out = _roi_align_sc(tbl, params,
                        jnp.asarray(_AY_CONST), jnp.asarray(_AX_CONST))
    return out.reshape(_N, _C, _OH, _OW)
